# Initial kernel scaffold; baseline (speedup 1.0000x reference)
import jax
import jax.numpy as jnp
from jax.experimental import pallas as pl


def kernel(input, pts_x1, pts_x2, indices_xj_i_cache):
    raise NotImplementedError("write your pallas kernel here")


# speedup vs baseline: 1.2112x; 1.2112x over previous
import functools

import jax
import jax.numpy as jnp
from jax import lax
from jax.experimental import pallas as pl
from jax.experimental.pallas import tpu as pltpu
from jax.experimental.pallas import tpu_sc as plsc

_K = 32
_GROUP = 4
_ROWS = _GROUP * _K
_LANES = 16


def _sc_maxpool(table, idx2, n_pad):
    n1, c = table.shape
    info = plsc.get_sparse_core_info()
    nw = info.num_cores * info.num_subcores
    tpw = n_pad // nw
    gpw = tpw // _GROUP

    mesh = plsc.VectorSubcoreMesh(core_axis_name="c", subcore_axis_name="s")

    @functools.partial(
        pl.kernel,
        mesh=mesh,
        out_type=jax.ShapeDtypeStruct((n_pad, c), jnp.float32),
        scratch_types=[
            pltpu.VMEM((gpw, _ROWS), jnp.int32),
            pltpu.VMEM((_ROWS, c), jnp.float32),
            pltpu.VMEM((tpw, c), jnp.float32),
            pltpu.SemaphoreType.DMA,
        ],
    )
    def k(table_hbm, idx_hbm, out_hbm, idx_v, gbuf, obuf, sem):
        w = lax.axis_index("s") * info.num_cores + lax.axis_index("c")
        pltpu.sync_copy(idx_hbm.at[pl.ds(w * gpw, gpw)], idx_v)

        @pl.loop(0, gpw)
        def _per_group(g):
            pltpu.async_copy(table_hbm.at[idx_v.at[g]], gbuf, sem).wait()

            @pl.loop(0, _GROUP)
            def _per_target(t):
                row = t * _K
                for j in range(c // _LANES):
                    sl = pl.ds(j * _LANES, _LANES)
                    acc = gbuf[row, sl]
                    for r in range(1, _K):
                        acc = jnp.maximum(acc, gbuf[row + r, sl])
                    obuf[g * _GROUP + t, sl] = acc

        pltpu.sync_copy(obuf, out_hbm.at[pl.ds(w * tpw, tpw)])

    return k(table, idx2)


def kernel(input, pts_x1, pts_x2, indices_xj_i_cache):
    u = input.reshape(-1, input.shape[-1])
    n2, k = indices_xj_i_cache.shape
    assert k == _K
    n_pad = -(-n2 // 1024) * 1024
    table = u.T
    idx = indices_xj_i_cache.astype(jnp.int32)
    idxp = jnp.pad(idx, ((0, n_pad - n2), (0, 0)))
    idx2 = idxp.reshape(n_pad * _K // _ROWS, _ROWS)
    out_t = _sc_maxpool(table, idx2, n_pad)
    res = out_t[:n2].T
    return (res.reshape(*input.shape[:-1], n2), pts_x2)


# speedup vs baseline: 1.3520x; 1.1163x over previous
import functools

import jax
import jax.numpy as jnp
from jax import lax
from jax.experimental import pallas as pl
from jax.experimental.pallas import tpu as pltpu
from jax.experimental.pallas import tpu_sc as plsc

_K = 32
_GROUP = 4
_ROWS = _GROUP * _K
_LANES = 16
_NBUF = 4


def _sc_maxpool(table, idx2, n_pad):
    n1, c = table.shape
    info = plsc.get_sparse_core_info()
    nw = info.num_cores * info.num_subcores
    tpw = n_pad // nw
    gpw = tpw // _GROUP

    mesh = plsc.VectorSubcoreMesh(core_axis_name="c", subcore_axis_name="s")

    @functools.partial(
        pl.kernel,
        mesh=mesh,
        out_type=jax.ShapeDtypeStruct((n_pad, c), jnp.float32),
        scratch_types=[
            pltpu.VMEM((gpw, _ROWS), jnp.int32),
            pltpu.VMEM((_NBUF, _ROWS, c), jnp.float32),
            pltpu.VMEM((tpw, c), jnp.float32),
            pltpu.SemaphoreType.DMA((_NBUF,)),
        ],
    )
    def k(table_hbm, idx_hbm, out_hbm, idx_v, gbuf, obuf, sem):
        w = lax.axis_index("s") * info.num_cores + lax.axis_index("c")
        pltpu.sync_copy(idx_hbm.at[pl.ds(w * gpw, gpw)], idx_v)

        def start_gather(g, b):
            pltpu.async_copy(table_hbm.at[idx_v.at[g]], gbuf.at[b], sem.at[b])

        def wait_gather(g, b):
            pltpu.make_async_copy(
                table_hbm.at[idx_v.at[g]], gbuf.at[b], sem.at[b]
            ).wait()

        for b in range(_NBUF):
            start_gather(b, b)

        @pl.loop(0, gpw, step=_NBUF)
        def _per_block(gg):
            for b in range(_NBUF):
                g = gg + b
                wait_gather(g, b)

                @pl.loop(0, _GROUP)
                def _per_target(t):
                    row = t * _K
                    for j in range(c // _LANES):
                        sl = pl.ds(j * _LANES, _LANES)
                        acc = gbuf[b, row, sl]
                        for r in range(1, _K):
                            acc = jnp.maximum(acc, gbuf[b, row + r, sl])
                        obuf[g * _GROUP + t, sl] = acc

                @pl.when(g + _NBUF < gpw)
                def _refill():
                    start_gather(g + _NBUF, b)

        pltpu.sync_copy(obuf, out_hbm.at[pl.ds(w * tpw, tpw)])

    return k(table, idx2)


def kernel(input, pts_x1, pts_x2, indices_xj_i_cache):
    u = input.reshape(-1, input.shape[-1])
    n2, k = indices_xj_i_cache.shape
    assert k == _K
    n_pad = -(-n2 // 1024) * 1024
    table = u.T
    idx = indices_xj_i_cache.astype(jnp.int32)
    idxp = jnp.pad(idx, ((0, n_pad - n2), (0, 0)))
    idx2 = idxp.reshape(n_pad * _K // _ROWS, _ROWS)
    out_t = _sc_maxpool(table, idx2, n_pad)
    res = out_t[:n2].T
    return (res.reshape(*input.shape[:-1], n2), pts_x2)


# speedup vs baseline: 6.0066x; 4.4427x over previous
import dataclasses
import functools

import jax
import jax.numpy as jnp
from jax import lax
from jax.experimental import pallas as pl
from jax.experimental.pallas import tpu as pltpu
from jax.experimental.pallas import tpu_sc as plsc

_K = 32
_LANES = 16
_CPW = 8
_TB = 512
_UB = _TB // _LANES


def _sc_maxpool(inp, idx_t, n_pad):
    c, n1 = inp.shape
    info = plsc.get_sparse_core_info()
    nc = info.num_cores
    nw = nc * info.num_subcores
    ncg = c // _CPW
    nh = nw // ncg
    ht = n_pad // nh
    nb = ht // _TB

    mesh = plsc.VectorSubcoreMesh(core_axis_name="c", subcore_axis_name="s")
    cp = pltpu.CompilerParams()
    if "needs_layout_passes" in pltpu.CompilerParams.__dataclass_fields__:
        cp = dataclasses.replace(cp, needs_layout_passes=False)

    @functools.partial(
        pl.kernel,
        mesh=mesh,
        compiler_params=cp,
        out_type=jax.ShapeDtypeStruct((c, n_pad), jnp.float32),
        scratch_types=[
            pltpu.VMEM((_CPW, n1), jnp.float32),
            pltpu.VMEM((2, _K, _TB), jnp.int32),
            pltpu.VMEM((2, _CPW, _TB), jnp.float32),
            pltpu.SemaphoreType.DMA,
            pltpu.SemaphoreType.DMA((2,)),
            pltpu.SemaphoreType.DMA((2,)),
        ],
    )
    def k(in_hbm, idx_hbm, out_hbm, tab_v, idx_v, obuf, tsem, isem, osem):
        w = lax.axis_index("s") * nc + lax.axis_index("c")
        cg = w % ncg
        half = w // ncg
        c0 = cg * _CPW
        h0 = half * ht

        cvecs = [jnp.full((_LANES,), cc, jnp.int32) for cc in range(_CPW)]

        def idx_copy(blk, p):
            return pltpu.make_async_copy(
                idx_hbm.at[:, pl.ds(h0 + blk * _TB, _TB)], idx_v.at[p],
                isem.at[p])

        def out_copy(blk, p):
            return pltpu.make_async_copy(
                obuf.at[p], out_hbm.at[pl.ds(c0, _CPW),
                                       pl.ds(h0 + blk * _TB, _TB)],
                osem.at[p])

        tab_cp = pltpu.make_async_copy(in_hbm.at[pl.ds(c0, _CPW)], tab_v, tsem)
        tab_cp.start()
        idx_copy(0, 0).start()
        idx_copy(1, 1).start()
        tab_cp.wait()

        @pl.loop(0, nb, step=2)
        def _per_pair(bi):
            for p in range(2):
                blk = bi + p
                idx_copy(blk, p).wait()

                @pl.when(blk >= 2)
                def _drain_store():
                    out_copy(blk, p).wait()

                @pl.loop(0, _UB)
                def _per_chunk(u):
                    sl = pl.ds(u * _LANES, _LANES)
                    iv = [idx_v[p, kk, sl] for kk in range(_K)]
                    for cc in range(_CPW):
                        acc = plsc.load_gather(tab_v, [cvecs[cc], iv[0]])
                        for kk in range(1, _K):
                            acc = jnp.maximum(
                                acc, plsc.load_gather(tab_v, [cvecs[cc], iv[kk]]))
                        obuf[p, cc, sl] = acc

                @pl.when(blk + 2 < nb)
                def _refill():
                    idx_copy(blk + 2, p).start()

                out_copy(blk, p).start()

        out_copy(0, 0).wait()
        out_copy(1, 1).wait()

    return k(inp, idx_t)


def kernel(input, pts_x1, pts_x2, indices_xj_i_cache):
    u = input.reshape(-1, input.shape[-1])
    n2, k = indices_xj_i_cache.shape
    assert k == _K
    n_pad = -(-n2 // 1024) * 1024
    idx = indices_xj_i_cache.astype(jnp.int32)
    idx_t = jnp.pad(idx, ((0, n_pad - n2), (0, 0))).T
    out = _sc_maxpool(u, idx_t, n_pad)
    return (out[:, :n2].reshape(*input.shape[:-1], n2), pts_x2)
